# baseline (speedup 1.0000x reference)
import functools

import jax
import jax.numpy as jnp
from jax import lax
from jax.experimental import pallas as pl
from jax.experimental.pallas import tpu as pltpu
from jax.experimental.pallas import tpu_sc as plsc

_K = 1024
_D = 256
_R = 512
_N = 16384
_N1 = 12288
_N2 = _N - _N1

_NC = 2
_NS = 16
_NW = _NC * _NS
_CH = 128

_SC_MESH = plsc.VectorSubcoreMesh(core_axis_name="c", subcore_axis_name="s")


def _vq_body(x_ref, et_ref, e2_ref, enc_ref, idx_ref, loss_ref):
    i = pl.program_id(0)
    x = x_ref[...]
    x2 = jnp.sum(x ** 2, axis=1, keepdims=True)
    m = jnp.dot(x, et_ref[...],
                preferred_element_type=jnp.float32)
    d = (x2 + e2_ref[...]) - 2.0 * m
    minv = jnp.min(d, axis=1, keepdims=True)
    iota_f = lax.broadcasted_iota(
        jnp.int32, (_R, _K), 1).astype(jnp.float32)
    idx_f = jnp.min(jnp.where(d == minv, iota_f, float(_K)), axis=1)
    enc_ref[...] = (iota_f == idx_f[:, None]).astype(jnp.float32)
    idx_ref[...] = idx_f.astype(jnp.int32)[:, None]

    @pl.when(i == 0)
    def _():
        loss_ref[...] = jnp.zeros((1, 1), jnp.float32)

    loss_ref[...] += jnp.sum(minv).reshape(1, 1)


def _vq_body2(x_ref, et_ref, e2_ref, enc_in_ref, enc_ref, idx_ref, loss_ref):
    del enc_in_ref
    _vq_body(x_ref, et_ref, e2_ref, enc_ref, idx_ref, loss_ref)


def _gather_chunks(emb_ref, idx_ref, out_ref, n_rows, idx_off, out_off):
    bw = n_rows // _NW
    nch = bw // _CH

    def scoped(idx_v, rows_v, sem):
        wid = lax.axis_index("s") * _NC + lax.axis_index("c")
        base = wid * bw
        for c in range(nch):
            off = base + c * _CH
            pltpu.sync_copy(idx_ref.at[pl.ds(idx_off + off, _CH)], idx_v)
            pltpu.async_copy(emb_ref.at[idx_v], rows_v, sem).wait()
            pltpu.sync_copy(rows_v, out_ref.at[pl.ds(out_off + off, _CH)])

    pl.run_scoped(
        scoped,
        pltpu.VMEM((_CH,), jnp.int32),
        pltpu.VMEM((_CH, _D), jnp.float32),
        pltpu.SemaphoreType.DMA,
    )


def _sc_gather1(emb_hbm, idx_hbm, out_hbm, idx_v, rows_v, sem):
    wid = lax.axis_index("s") * _NC + lax.axis_index("c")
    bw = _N1 // _NW
    base = wid * bw
    for c in range(bw // _CH):
        off = base + c * _CH
        pltpu.sync_copy(idx_hbm.at[pl.ds(off, _CH)], idx_v)
        pltpu.async_copy(emb_hbm.at[idx_v], rows_v, sem).wait()
        pltpu.sync_copy(rows_v, out_hbm.at[pl.ds(off, _CH)])


def kernel(inputs, embedding):
    input_shape = inputs.shape
    flat = inputs.reshape(-1, _D)
    e2 = jnp.sum(embedding ** 2, axis=1)[None, :]
    et = embedding.T

    g1 = _N1 // _R
    g2 = _N2 // _R
    enc_part, idx_a, loss_a = pl.pallas_call(
        _vq_body,
        grid=(g1,),
        in_specs=[
            pl.BlockSpec((_R, _D), lambda i: (i, 0)),
            pl.BlockSpec((_D, _K), lambda i: (0, 0)),
            pl.BlockSpec((1, _K), lambda i: (0, 0)),
        ],
        out_specs=[
            pl.BlockSpec((_R, _K), lambda i: (i, 0)),
            pl.BlockSpec((_R, 1), lambda i: (i, 0)),
            pl.BlockSpec((1, 1), lambda i: (0, 0)),
        ],
        out_shape=[
            jax.ShapeDtypeStruct((_N, _K), jnp.float32),
            jax.ShapeDtypeStruct((_N1, 1), jnp.int32),
            jax.ShapeDtypeStruct((1, 1), jnp.float32),
        ],
        compiler_params=pltpu.CompilerParams(
            dimension_semantics=("arbitrary",),
        ),
    )(flat, et, e2)

    gather1 = functools.partial(
        pl.kernel,
        mesh=_SC_MESH,
        out_type=jax.ShapeDtypeStruct((_N, _D), jnp.float32),
        scratch_types=[
            pltpu.VMEM((_CH,), jnp.int32),
            pltpu.VMEM((_CH, _D), jnp.float32),
            pltpu.SemaphoreType.DMA,
        ],
    )(_sc_gather1)
    q_a = gather1(embedding, idx_a.reshape(_N1))

    enc, idx_b, loss_b = pl.pallas_call(
        _vq_body2,
        grid=(g2,),
        in_specs=[
            pl.BlockSpec((_R, _D), lambda i: (i + g1, 0)),
            pl.BlockSpec((_D, _K), lambda i: (0, 0)),
            pl.BlockSpec((1, _K), lambda i: (0, 0)),
            pl.BlockSpec((8, 128), lambda i: (0, 0)),
        ],
        out_specs=[
            pl.BlockSpec((_R, _K), lambda i: (i + g1, 0)),
            pl.BlockSpec((_R, 1), lambda i: (i, 0)),
            pl.BlockSpec((1, 1), lambda i: (0, 0)),
        ],
        out_shape=[
            jax.ShapeDtypeStruct((_N, _K), jnp.float32),
            jax.ShapeDtypeStruct((_N2, 1), jnp.int32),
            jax.ShapeDtypeStruct((1, 1), jnp.float32),
        ],
        input_output_aliases={3: 0},
        compiler_params=pltpu.CompilerParams(
            dimension_semantics=("arbitrary",),
        ),
    )(flat, et, e2, enc_part)

    def _finish_q(refs):
        emb_ref, idxb_ref, q_ref = refs

        @pl.core_map(_SC_MESH)
        def _():
            _gather_chunks(emb_ref, idxb_ref, q_ref, _N2,
                           idx_off=0, out_off=_N1)

    _, _, q = pl.run_state(_finish_q)(
        (embedding, idx_b.reshape(_N2), q_a))

    total = loss_a[0, 0] + loss_b[0, 0]
    mean_sq = total / (_N * _D)
    loss = mean_sq + 0.25 * mean_sq
    quantized = q.reshape(input_shape)
    encoding_indices = jnp.concatenate(
        [idx_a, idx_b], axis=0).reshape(input_shape[:-1])
    return (quantized, loss, enc, encoding_indices)


# speedup vs baseline: 1.2084x; 1.2084x over previous
import functools

import jax
import jax.numpy as jnp
from jax import lax
from jax.experimental import pallas as pl
from jax.experimental.pallas import tpu as pltpu
from jax.experimental.pallas import tpu_sc as plsc

_K = 1024
_D = 256
_R = 2048
_N = 16384

_NC = 2
_NS = 16
_NW = _NC * _NS
_BW = _N // _NW
_CH = 128
_NCH = _BW // _CH


def _vq_block(x_ref, et_ref, e2_ref,
              enc_ref, idx_ref, loss_ref):
    i = pl.program_id(0)
    x = x_ref[...]
    x2 = jnp.sum(x ** 2, axis=1, keepdims=True)
    m = jnp.dot(x, et_ref[...],
                preferred_element_type=jnp.float32)
    d = (x2 + e2_ref[...]) - 2.0 * m
    minv = jnp.min(d, axis=1, keepdims=True)
    iota_f = lax.broadcasted_iota(
        jnp.int32, (_R, _K), 1).astype(jnp.float32)
    idx_f = jnp.min(jnp.where(d == minv, iota_f, float(_K)), axis=1)
    enc_ref[...] = (iota_f == idx_f[:, None]).astype(jnp.float32)
    idx_ref[...] = idx_f.astype(jnp.int32)[:, None]

    @pl.when(i == 0)
    def _():
        loss_ref[...] = jnp.zeros((1, 1), jnp.float32)

    loss_ref[...] += jnp.sum(minv).reshape(1, 1)


def _sc_gather(emb_hbm, idx_hbm, out_hbm, idx_v, rows_v, sem):
    wid = lax.axis_index("s") * _NC + lax.axis_index("c")
    base = wid * _BW
    for c in range(_NCH):
        off = base + c * _CH
        pltpu.sync_copy(idx_hbm.at[pl.ds(off, _CH)], idx_v)
        pltpu.async_copy(emb_hbm.at[idx_v], rows_v, sem).wait()
        pltpu.sync_copy(rows_v, out_hbm.at[pl.ds(off, _CH)])


def kernel(inputs, embedding):
    input_shape = inputs.shape
    flat = inputs.reshape(-1, _D)
    e2 = jnp.sum(embedding ** 2, axis=1)[None, :]
    et = embedding.T

    grid = _N // _R
    enc, idx3, loss_acc = pl.pallas_call(
        _vq_block,
        grid=(grid,),
        in_specs=[
            pl.BlockSpec((_R, _D), lambda i: (i, 0)),
            pl.BlockSpec((_D, _K), lambda i: (0, 0)),
            pl.BlockSpec((1, _K), lambda i: (0, 0)),
        ],
        out_specs=[
            pl.BlockSpec((_R, _K), lambda i: (i, 0)),
            pl.BlockSpec((_R, 1), lambda i: (i, 0)),
            pl.BlockSpec((1, 1), lambda i: (0, 0)),
        ],
        out_shape=[
            jax.ShapeDtypeStruct((_N, _K), jnp.float32),
            jax.ShapeDtypeStruct((_N, 1), jnp.int32),
            jax.ShapeDtypeStruct((1, 1), jnp.float32),
        ],
        compiler_params=pltpu.CompilerParams(
            dimension_semantics=("arbitrary",),
        ),
    )(flat, et, e2)

    idx_flat = idx3.reshape(_N)
    gather = functools.partial(
        pl.kernel,
        mesh=plsc.VectorSubcoreMesh(core_axis_name="c", subcore_axis_name="s"),
        out_type=jax.ShapeDtypeStruct((_N, _D), jnp.float32),
        scratch_types=[
            pltpu.VMEM((_CH,), jnp.int32),
            pltpu.VMEM((_CH, _D), jnp.float32),
            pltpu.SemaphoreType.DMA,
        ],
    )(_sc_gather)
    q = gather(embedding, idx_flat)

    mean_sq = loss_acc[0, 0] / (_N * _D)
    loss = mean_sq + 0.25 * mean_sq
    quantized = q.reshape(input_shape)
    encoding_indices = idx3.reshape(input_shape[:-1])
    return (quantized, loss, enc, encoding_indices)
